# baseline (speedup 1.0000x reference)
import functools

import jax
import jax.numpy as jnp
from jax import lax
from jax.experimental import pallas as pl
from jax.experimental.pallas import tpu as pltpu
from jax.experimental.pallas import tpu_sc as plsc


def _build(seq, feat, seqs_per_w, nw, nc):
    half = seq // 2
    mesh = plsc.VectorSubcoreMesh(core_axis_name="c", subcore_axis_name="s")

    @functools.partial(
        pl.kernel,
        mesh=mesh,
        out_type=jax.ShapeDtypeStruct((nw, seqs_per_w, seq, feat), jnp.float32),
        scratch_types=[
            pltpu.VMEM((seqs_per_w, 2, half), jnp.int32),
            pltpu.VMEM((seq, feat), jnp.float32),
            pltpu.VMEM((seq, feat), jnp.float32),
            pltpu.SemaphoreType.DMA,
        ],
        compiler_params=pltpu.CompilerParams(use_tc_tiling_on_sc=False),
    )
    def emb_kernel(idx_hbm, tok_hbm, pos_hbm, out_hbm, idx_v, pos_v, rows_v, gsem):
        wid = lax.axis_index("s") * nc + lax.axis_index("c")
        pltpu.sync_copy(idx_hbm.at[wid], idx_v)
        pltpu.sync_copy(pos_hbm, pos_v)

        def chunk(c, carry):
            ga = pltpu.async_copy(
                tok_hbm.at[idx_v.at[c, 0]], rows_v.at[pl.ds(0, half)], gsem
            )
            gb = pltpu.async_copy(
                tok_hbm.at[idx_v.at[c, 1]], rows_v.at[pl.ds(half, half)], gsem
            )
            ga.wait()
            gb.wait()

            def srow(s, inner):
                for j in range(feat // 16):
                    sl = pl.ds(j * 16, 16)
                    plsc.addupdate(rows_v.at[s, sl], pos_v[s, sl])
                return inner

            lax.fori_loop(0, seq, srow, 0)
            pltpu.sync_copy(rows_v, out_hbm.at[wid, c])
            return carry

        lax.fori_loop(0, seqs_per_w, chunk, 0)

    return emb_kernel


def kernel(inputs, token_table, pos_table):
    batch, seq = inputs.shape
    feat = token_table.shape[1]
    info = plsc.get_sparse_core_info()
    nc, ns = info.num_cores, info.num_subcores
    nw = nc * ns
    total = batch * seq
    rows_per_w = total // nw
    seqs_per_w = rows_per_w // seq

    emb = _build(seq, feat, seqs_per_w, nw, nc)
    idx = inputs.astype(jnp.int32).reshape(nw, seqs_per_w, 2, seq // 2)
    out = emb(idx, token_table, pos_table)
    return out.reshape(batch, seq, feat)


# speedup vs baseline: 1.1488x; 1.1488x over previous
import functools

import jax
import jax.numpy as jnp
from jax import lax
from jax.experimental import pallas as pl
from jax.experimental.pallas import tpu as pltpu
from jax.experimental.pallas import tpu_sc as plsc


_NBUF = 4


def _build(seq, feat, seqs_per_w, nw, nc):
    half = seq // 2
    nbuf = _NBUF
    mesh = plsc.VectorSubcoreMesh(core_axis_name="c", subcore_axis_name="s")

    @functools.partial(
        pl.kernel,
        mesh=mesh,
        out_type=jax.ShapeDtypeStruct((nw, seqs_per_w, seq, feat), jnp.float32),
        scratch_types=[
            pltpu.VMEM((seqs_per_w, 2, half), jnp.int32),
            pltpu.VMEM((seq, feat), jnp.float32),
            pltpu.VMEM((nbuf, seq, feat), jnp.float32),
            pltpu.SemaphoreType.DMA,
            pltpu.SemaphoreType.DMA,
        ],
        compiler_params=pltpu.CompilerParams(use_tc_tiling_on_sc=False),
    )
    def emb_kernel(idx_hbm, tok_hbm, pos_hbm, out_hbm, idx_v, pos_v, rows_v, gsem, osem):
        wid = lax.axis_index("s") * nc + lax.axis_index("c")
        pltpu.sync_copy(idx_hbm.at[wid], idx_v)
        pltpu.sync_copy(pos_hbm, pos_v)

        def start_gather(c, b):
            pltpu.async_copy(
                tok_hbm.at[idx_v.at[c, 0]], rows_v.at[b, pl.ds(0, half)], gsem
            )
            pltpu.async_copy(
                tok_hbm.at[idx_v.at[c, 1]], rows_v.at[b, pl.ds(half, half)], gsem
            )

        def wait_gather(b):
            pltpu.make_async_copy(
                tok_hbm.at[pl.ds(0, seq)], rows_v.at[b], gsem
            ).wait()

        def start_out(c, b):
            pltpu.async_copy(rows_v.at[b], out_hbm.at[wid, c], osem)

        def wait_out(b):
            pltpu.make_async_copy(
                out_hbm.at[wid, 0], rows_v.at[b], osem
            ).wait()

        start_gather(0, 0)
        start_gather(1, 1)

        def outer(g, carry):
            for b in range(nbuf):
                c = g * nbuf + b
                wait_gather(b)

                def srow(s, inner):
                    for j in range(feat // 16):
                        sl = pl.ds(j * 16, 16)
                        plsc.addupdate(rows_v.at[b, s, sl], pos_v[s, sl])
                    return inner

                lax.fori_loop(0, seq, srow, 0)
                start_out(c, b)

                bn = (b + 2) % nbuf

                @pl.when(c >= 2)
                def _():
                    wait_out(bn)

                @pl.when(c + 2 < seqs_per_w)
                def _():
                    start_gather(c + 2, bn)

            return carry

        lax.fori_loop(0, seqs_per_w // nbuf, outer, 0)
        wait_out((seqs_per_w - 2) % nbuf)
        wait_out((seqs_per_w - 1) % nbuf)

    return emb_kernel


def kernel(inputs, token_table, pos_table):
    batch, seq = inputs.shape
    feat = token_table.shape[1]
    info = plsc.get_sparse_core_info()
    nc, ns = info.num_cores, info.num_subcores
    nw = nc * ns
    total = batch * seq
    rows_per_w = total // nw
    seqs_per_w = rows_per_w // seq

    emb = _build(seq, feat, seqs_per_w, nw, nc)
    idx = inputs.astype(jnp.int32).reshape(nw, seqs_per_w, 2, seq // 2)
    out = emb(idx, token_table, pos_table)
    return out.reshape(batch, seq, feat)
